# baseline (speedup 1.0000x reference)
import functools

import jax
import jax.numpy as jnp
import numpy as np
from jax import lax
from jax.experimental import pallas as pl
from jax.experimental.pallas import tpu as pltpu
from jax.experimental.pallas import tpu_sc as plsc

VOCAB = 1000000
D_MODEL = 64
SEQ = 200
BATCH = 1024

NUM_CORES = 2
NUM_SUBCORES = 16
NW = NUM_CORES * NUM_SUBCORES
QB = 128
NQ = BATCH // QB
UNITS = SEQ * NQ
UPW = UNITS // NW
NBUF = 4
PIPE = 3

CH = 8192
N_IN_BLOCKS = -(-VOCAB // CH)
VOCAB_P = ((N_IN_BLOCKS + 1) // 2) * CH


def _positional_encoding(length, depth):
    depth_h = depth / 2
    positions = np.arange(length)[:, np.newaxis]
    depths = np.arange(depth_h)[np.newaxis, :] / depth_h
    angle_rates = 1 / 10000 ** depths
    angle_rads = positions * angle_rates
    return np.concatenate(
        [np.sin(angle_rads), np.cos(angle_rads)], axis=-1
    ).astype(np.float32)


_POS_SPLAT_NP = np.zeros((SEQ + 8, 16 * D_MODEL), np.float32)
_POS_SPLAT_NP[:SEQ] = np.repeat(
    _positional_encoding(SEQ, D_MODEL), 16, axis=1
)


def _tc_pack_kernel(t_ref, out_ref):
    h = CH // 2
    a1 = jnp.transpose(t_ref[:, 0:h], (1, 0))
    ii = lax.broadcasted_iota(jnp.int32, (D_MODEL, D_MODEL), 0)
    jj = lax.broadcasted_iota(jnp.int32, (D_MODEL, D_MODEL), 1)
    ident = (ii == jj).astype(jnp.float32)
    dn = (((0,), (0,)), ((), ()))
    a2 = lax.dot_general(t_ref[:, h:CH], ident, dn,
                         preferred_element_type=jnp.float32)
    i = pl.program_id(0)

    @pl.when(i % 2 == 0)
    def _():
        out_ref[0:h, 0:D_MODEL] = a1
        out_ref[h:CH, 0:D_MODEL] = a2

    @pl.when(i % 2 == 1)
    def _():
        out_ref[0:h, D_MODEL:2 * D_MODEL] = a1
        out_ref[h:CH, D_MODEL:2 * D_MODEL] = a2


def _pack_table(tT):
    return pl.pallas_call(
        _tc_pack_kernel,
        grid=(N_IN_BLOCKS,),
        in_specs=[pl.BlockSpec((D_MODEL, CH), lambda i: (0, i))],
        out_specs=pl.BlockSpec((CH, 2 * D_MODEL), lambda i: (i // 2, 0)),
        out_shape=jax.ShapeDtypeStruct((VOCAB_P, 2 * D_MODEL), jnp.float32),
    )(tT)


def _sc_kernel(tableP, xT, posS, outT, idx_v, pairs_v, hoff_v, g_v, pos_v,
               out_v, sem_i, sem_g, sem_o):
    wid = lax.axis_index("s") * NUM_CORES + lax.axis_index("c")
    u0 = wid * UPW
    p0a = ((u0 // NQ) // 8) * 8
    iota = lax.iota(jnp.int32, 16)
    scale = jnp.float32(8.0)

    pltpu.sync_copy(posS.at[pl.ds(p0a, 16)], pos_v)

    def pq(k):
        u = u0 + k
        return u // NQ, u % NQ

    def idx_start(k):
        kc = jnp.minimum(k, UPW - 1)
        p, q = pq(kc)
        p8 = (p // 8) * 8
        b = lax.rem(k, NBUF)
        pltpu.async_copy(
            xT.at[pl.ds(p8, 8), pl.ds(q * QB, QB)],
            idx_v.at[pl.ds(b * 8, 8)],
            sem_i.at[b],
        )

    def prep_and_gather_start(k):
        b = lax.rem(k, NBUF)
        p, _ = pq(k)
        row = b * 8 + lax.rem(p, 8)

        def prep(i, _):
            rsl = pl.ds(i * 16, 16)
            wsl = pl.ds(b * QB + i * 16, 16)
            raw = idx_v[row, rsl]
            pairs_v[wsl] = lax.bitwise_or(
                lax.shift_left(lax.shift_right_logical(raw, 14), 13),
                lax.bitwise_and(raw, 8191),
            )
            hoff_v[wsl] = lax.shift_right_logical(
                lax.bitwise_and(raw, 8192), 7
            )
            return 0

        lax.fori_loop(0, QB // 16, prep, 0)
        pltpu.async_copy(
            tableP.at[pairs_v.at[pl.ds(b * QB, QB)]],
            g_v.at[pl.ds(b * QB, QB)],
            sem_g.at[b],
        )

    def compute(k):
        bg = lax.rem(k, NBUF)
        bo = lax.rem(k, 2)
        p, _ = pq(k)
        pi = p - p0a

        def group(g, _):
            rbase = bg * QB + g * 16
            rvec = iota + rbase
            hsl = hoff_v[pl.ds(rbase, 16)]
            for d in range(D_MODEL):
                cvec = hsl + d
                val = plsc.load_gather(g_v, [rvec, cvec])
                out_v[bo * D_MODEL + d, pl.ds(g * 16, 16)] = (
                    val * scale + pos_v[pi, pl.ds(d * 16, 16)]
                )
            return 0

        lax.fori_loop(0, QB // 16, group, 0)

    def out_start(k):
        b = lax.rem(k, 2)
        p, q = pq(k)
        pltpu.async_copy(
            out_v.at[pl.ds(b * D_MODEL, D_MODEL)],
            outT.at[p, :, pl.ds(q * QB, QB)],
            sem_o.at[b],
        )

    def wait_i(b):
        pltpu.make_async_copy(
            xT.at[pl.ds(0, 8), pl.ds(0, QB)], idx_v.at[pl.ds(0, 8)],
            sem_i.at[b],
        ).wait()

    def wait_g(b):
        pltpu.make_async_copy(
            tableP.at[pl.ds(0, QB)], g_v.at[pl.ds(0, QB)], sem_g.at[b]
        ).wait()

    def wait_o(b):
        pltpu.make_async_copy(
            out_v.at[pl.ds(0, D_MODEL)], outT.at[0, :, pl.ds(0, QB)],
            sem_o.at[b],
        ).wait()

    for j in range(NBUF):
        idx_start(jnp.int32(j))
    for j in range(PIPE):
        wait_i(jnp.int32(j))
        prep_and_gather_start(jnp.int32(j))

    def stage(k, drain_out):
        b4 = lax.rem(k + PIPE, NBUF)
        if drain_out:
            wait_o(lax.rem(k, 2))
        wait_i(b4)
        prep_and_gather_start(k + PIPE)
        idx_start(k + NBUF)
        wait_g(lax.rem(k, NBUF))
        compute(k)
        out_start(k)

    stage(jnp.int32(0), False)
    stage(jnp.int32(1), False)

    def body(k, _):
        stage(k, True)
        return 0

    lax.fori_loop(2, UPW - PIPE, body, 0)

    for j in range(PIPE, 0, -1):
        kl = jnp.int32(UPW - j)
        wait_o(lax.rem(kl, 2))
        wait_g(lax.rem(kl, NBUF))
        compute(kl)
        out_start(kl)
    wait_i(lax.rem(jnp.int32(UPW), NBUF))
    wait_o(lax.rem(jnp.int32(UPW - 2), 2))
    wait_o(lax.rem(jnp.int32(UPW - 1), 2))


@jax.jit
def _run(x, table):
    tT = table.T
    tableP = _pack_table(tT)
    xT = x.T
    mesh = plsc.VectorSubcoreMesh(core_axis_name="c", subcore_axis_name="s")
    k = functools.partial(
        pl.kernel,
        out_type=jax.ShapeDtypeStruct((SEQ, D_MODEL, BATCH), jnp.float32),
        mesh=mesh,
        scratch_types=[
            pltpu.VMEM((NBUF * 8, QB), jnp.int32),
            pltpu.VMEM((NBUF * QB,), jnp.int32),
            pltpu.VMEM((NBUF * QB,), jnp.int32),
            pltpu.VMEM((NBUF * QB, 2 * D_MODEL), jnp.float32),
            pltpu.VMEM((16, 16 * D_MODEL), jnp.float32),
            pltpu.VMEM((2 * D_MODEL, QB), jnp.float32),
            pltpu.SemaphoreType.DMA((NBUF,)),
            pltpu.SemaphoreType.DMA((NBUF,)),
            pltpu.SemaphoreType.DMA((2,)),
        ],
        compiler_params=pltpu.CompilerParams(needs_layout_passes=False),
    )(_sc_kernel)
    outT = k(tableP, xT, jnp.asarray(_POS_SPLAT_NP))
    return outT.transpose(2, 0, 1)


def kernel(x, table):
    return _run(x, table)


# speedup vs baseline: 1.0169x; 1.0169x over previous
import functools

import jax
import jax.numpy as jnp
import numpy as np
from jax import lax
from jax.experimental import pallas as pl
from jax.experimental.pallas import tpu as pltpu
from jax.experimental.pallas import tpu_sc as plsc

VOCAB = 1000000
D_MODEL = 64
SEQ = 200
BATCH = 1024

NUM_CORES = 2
NUM_SUBCORES = 16
NW = NUM_CORES * NUM_SUBCORES
QB = 128
NQ = BATCH // QB
UNITS = SEQ * NQ
UPW = UNITS // NW
NBUF = 4
PIPE = 3

CH = 8192
N_IN_BLOCKS = -(-VOCAB // CH)
VOCAB_P = ((N_IN_BLOCKS + 1) // 2) * CH


def _positional_encoding(length, depth):
    depth_h = depth / 2
    positions = np.arange(length)[:, np.newaxis]
    depths = np.arange(depth_h)[np.newaxis, :] / depth_h
    angle_rates = 1 / 10000 ** depths
    angle_rads = positions * angle_rates
    return np.concatenate(
        [np.sin(angle_rads), np.cos(angle_rads)], axis=-1
    ).astype(np.float32)


_POS_SPLAT_NP = np.zeros((SEQ + 8, 16 * D_MODEL), np.float32)
_POS_SPLAT_NP[:SEQ] = np.repeat(
    _positional_encoding(SEQ, D_MODEL), 16, axis=1
)


def _tc_pack_kernel(t_ref, out_ref):
    a = jnp.transpose(t_ref[...], (1, 0))
    i = pl.program_id(0)

    @pl.when(i % 2 == 0)
    def _():
        out_ref[:, 0:D_MODEL] = a

    @pl.when(i % 2 == 1)
    def _():
        out_ref[:, D_MODEL:2 * D_MODEL] = a


def _pack_table(tT):
    return pl.pallas_call(
        _tc_pack_kernel,
        grid=(N_IN_BLOCKS,),
        in_specs=[pl.BlockSpec((D_MODEL, CH), lambda i: (0, i))],
        out_specs=pl.BlockSpec((CH, 2 * D_MODEL), lambda i: (i // 2, 0)),
        out_shape=jax.ShapeDtypeStruct((VOCAB_P, 2 * D_MODEL), jnp.float32),
    )(tT)


def _sc_kernel(tableP, xT, posS, outT, idx_v, pairs_v, hoff_v, g_v, pos_v,
               out_v, sem_i, sem_g, sem_o):
    wid = lax.axis_index("s") * NUM_CORES + lax.axis_index("c")
    u0 = wid * UPW
    p0a = ((u0 // NQ) // 8) * 8
    iota = lax.iota(jnp.int32, 16)
    scale = jnp.float32(8.0)

    pltpu.sync_copy(posS.at[pl.ds(p0a, 16)], pos_v)

    def pq(k):
        u = u0 + k
        return u // NQ, u % NQ

    def idx_start(k):
        kc = jnp.minimum(k, UPW - 1)
        p, q = pq(kc)
        p8 = (p // 8) * 8
        b = lax.rem(k, NBUF)
        pltpu.async_copy(
            xT.at[pl.ds(p8, 8), pl.ds(q * QB, QB)],
            idx_v.at[pl.ds(b * 8, 8)],
            sem_i.at[b],
        )

    def prep_and_gather_start(k):
        b = lax.rem(k, NBUF)
        p, _ = pq(k)
        row = b * 8 + lax.rem(p, 8)

        def prep(i, _):
            rsl = pl.ds(i * 16, 16)
            wsl = pl.ds(b * QB + i * 16, 16)
            raw = idx_v[row, rsl]
            pairs_v[wsl] = lax.bitwise_or(
                lax.shift_left(lax.shift_right_logical(raw, 14), 13),
                lax.bitwise_and(raw, 8191),
            )
            hoff_v[wsl] = lax.shift_right_logical(
                lax.bitwise_and(raw, 8192), 7
            )
            return 0

        lax.fori_loop(0, QB // 16, prep, 0)
        pltpu.async_copy(
            tableP.at[pairs_v.at[pl.ds(b * QB, QB)]],
            g_v.at[pl.ds(b * QB, QB)],
            sem_g.at[b],
        )

    def compute(k):
        bg = lax.rem(k, NBUF)
        bo = lax.rem(k, 2)
        p, _ = pq(k)
        pi = p - p0a
        ob = bo * D_MODEL

        rvecs = [iota + (bg * QB + g * 16) for g in range(QB // 16)]
        hsls = [hoff_v[pl.ds(bg * QB + g * 16, 16)] for g in range(QB // 16)]

        def dloop(d, _):
            pv = pos_v[pi, pl.ds(d * 16, 16)]
            orow = ob + d
            for g in range(QB // 16):
                val = plsc.load_gather(g_v, [rvecs[g], hsls[g] + d])
                out_v[orow, pl.ds(g * 16, 16)] = val * scale + pv
            return 0

        lax.fori_loop(0, D_MODEL, dloop, 0)

    def out_start(k):
        b = lax.rem(k, 2)
        p, q = pq(k)
        pltpu.async_copy(
            out_v.at[pl.ds(b * D_MODEL, D_MODEL)],
            outT.at[p, :, pl.ds(q * QB, QB)],
            sem_o.at[b],
        )

    def wait_i(b):
        pltpu.make_async_copy(
            xT.at[pl.ds(0, 8), pl.ds(0, QB)], idx_v.at[pl.ds(0, 8)],
            sem_i.at[b],
        ).wait()

    def wait_g(b):
        pltpu.make_async_copy(
            tableP.at[pl.ds(0, QB)], g_v.at[pl.ds(0, QB)], sem_g.at[b]
        ).wait()

    def wait_o(b):
        pltpu.make_async_copy(
            out_v.at[pl.ds(0, D_MODEL)], outT.at[0, :, pl.ds(0, QB)],
            sem_o.at[b],
        ).wait()

    for j in range(NBUF):
        idx_start(jnp.int32(j))
    for j in range(PIPE):
        wait_i(jnp.int32(j))
        prep_and_gather_start(jnp.int32(j))

    def stage(k, drain_out):
        b4 = lax.rem(k + PIPE, NBUF)
        if drain_out:
            wait_o(lax.rem(k, 2))
        wait_i(b4)
        prep_and_gather_start(k + PIPE)
        idx_start(k + NBUF)
        wait_g(lax.rem(k, NBUF))
        compute(k)
        out_start(k)

    stage(jnp.int32(0), False)
    stage(jnp.int32(1), False)

    def body(k, _):
        stage(k, True)
        return 0

    lax.fori_loop(2, UPW - PIPE, body, 0)

    for j in range(PIPE, 0, -1):
        kl = jnp.int32(UPW - j)
        wait_o(lax.rem(kl, 2))
        wait_g(lax.rem(kl, NBUF))
        compute(kl)
        out_start(kl)
    wait_i(lax.rem(jnp.int32(UPW), NBUF))
    wait_o(lax.rem(jnp.int32(UPW - 2), 2))
    wait_o(lax.rem(jnp.int32(UPW - 1), 2))


@jax.jit
def _run(x, table):
    tT = table.T
    tableP = _pack_table(tT)
    xT = x.T
    mesh = plsc.VectorSubcoreMesh(core_axis_name="c", subcore_axis_name="s")
    k = functools.partial(
        pl.kernel,
        out_type=jax.ShapeDtypeStruct((SEQ, D_MODEL, BATCH), jnp.float32),
        mesh=mesh,
        scratch_types=[
            pltpu.VMEM((NBUF * 8, QB), jnp.int32),
            pltpu.VMEM((NBUF * QB,), jnp.int32),
            pltpu.VMEM((NBUF * QB,), jnp.int32),
            pltpu.VMEM((NBUF * QB, 2 * D_MODEL), jnp.float32),
            pltpu.VMEM((16, 16 * D_MODEL), jnp.float32),
            pltpu.VMEM((2 * D_MODEL, QB), jnp.float32),
            pltpu.SemaphoreType.DMA((NBUF,)),
            pltpu.SemaphoreType.DMA((NBUF,)),
            pltpu.SemaphoreType.DMA((2,)),
        ],
        compiler_params=pltpu.CompilerParams(needs_layout_passes=False),
    )(_sc_kernel)
    outT = k(tableP, xT, jnp.asarray(_POS_SPLAT_NP))
    return outT.transpose(2, 0, 1)


def kernel(x, table):
    return _run(x, table)
